# baseline (speedup 1.0000x reference)
import jax
import jax.numpy as jnp
import numpy as np
from jax.experimental import pallas as pl
from jax.experimental.pallas import tpu as pltpu

_NR = 8
_NW = 8
_H = 48
_NB = _NR * _NW
_N = 2048
_FC_OUT = 128
_BI = 16
_BR = 256


def _bin_indices(dx, dy):
    r = jnp.sqrt(dx * dx + dy * dy)
    ring = jnp.ceil((_NR - 1) * (jnp.log(r / 0.5) / 3.0))
    ring = jnp.where(jnp.isneginf(ring), 0.0, ring).astype(jnp.int32)

    ax = jnp.abs(dx)
    ay = jnp.abs(dy)
    ge = ay >= ax
    gt = ay > ax
    dxpos = dx > 0
    dxneg = dx < 0
    sbx = jnp.signbit(dx)
    sby = jnp.signbit(dy)
    w_pos = jnp.where(dxpos, jnp.where(ge, 4, 3), jnp.where(gt, 5, 6))
    w_neg = jnp.where(dxneg, 0, jnp.where(ay <= ax, 2, 1))
    w_zero = jnp.where(sbx, jnp.where(sby, -1, 7), 3)
    wedge = jnp.where(dy > 0, w_pos, jnp.where(dy < 0, w_neg, w_zero))

    fin = ring * _NW + wedge
    fin = jnp.where(fin < _NW, 0, fin)
    fin = jnp.where(fin >= _NW * _NW, 0, fin)
    return fin


def _means_step(yx_col, yy_col, yx_row, yy_row, hid_hi, hid_lo, m_ref):
    dx = yx_row[...] - yx_col[...]
    dy = yy_row[...] - yy_col[...]
    fin = _bin_indices(dx, dy)

    k_iota = jax.lax.broadcasted_iota(jnp.int32, (_BI, _NB, _N), 1)
    onehot = (fin[:, None, :] == k_iota + _NW).astype(jnp.bfloat16)
    onehot = onehot.reshape(_BI * _NB, _N)

    s = (jnp.dot(onehot, hid_hi[...], preferred_element_type=jnp.float32)
         + jnp.dot(onehot, hid_lo[...], preferred_element_type=jnp.float32))
    cnt = s[:, _H:_H + 1]
    recip = 1.0 / jnp.maximum(cnt, 1.0)
    m_ref[...] = s[:, :_H] * recip


def _fc_step(m, Wt, b, out_ref):
    o = jnp.dot(m[...], Wt[...], preferred_element_type=jnp.float32) + b[...]
    out_ref[...] = jnp.maximum(o, 0.0)


def kernel(y_pred, x_start, hidden, W, b):
    del x_start
    yx_col = y_pred[:, 0:1]
    yy_col = y_pred[:, 1:2]
    yx_row = y_pred[:, 0].reshape(1, _N)
    yy_row = y_pred[:, 1].reshape(1, _N)
    hidden_aug = jnp.concatenate(
        [hidden, jnp.ones((_N, 1), hidden.dtype),
         jnp.zeros((_N, 64 - _H - 1), hidden.dtype)], axis=1)
    hid_hi = hidden_aug.astype(jnp.bfloat16)
    hid_lo = (hidden_aug - hid_hi.astype(jnp.float32)).astype(jnp.bfloat16)

    means = pl.pallas_call(
        _means_step,
        grid=(_N // _BI,),
        in_specs=[
            pl.BlockSpec((_BI, 1), lambda i: (i, 0)),
            pl.BlockSpec((_BI, 1), lambda i: (i, 0)),
            pl.BlockSpec((1, _N), lambda i: (0, 0)),
            pl.BlockSpec((1, _N), lambda i: (0, 0)),
            pl.BlockSpec((_N, 64), lambda i: (0, 0)),
            pl.BlockSpec((_N, 64), lambda i: (0, 0)),
        ],
        out_specs=pl.BlockSpec((_BI * _NB, _H), lambda i: (i, 0)),
        out_shape=jax.ShapeDtypeStruct((_N * _NB, _H), jnp.float32),
    )(yx_col, yy_col, yx_row, yy_row, hid_hi, hid_lo)

    m2 = means.reshape(_N, _NB * _H)

    Wt = W.T
    b2 = b.reshape(1, _FC_OUT)
    return pl.pallas_call(
        _fc_step,
        grid=(_N // _BR,),
        in_specs=[
            pl.BlockSpec((_BR, _NB * _H), lambda i: (i, 0)),
            pl.BlockSpec((_NB * _H, _FC_OUT), lambda i: (0, 0)),
            pl.BlockSpec((1, _FC_OUT), lambda i: (0, 0)),
        ],
        out_specs=pl.BlockSpec((_BR, _FC_OUT), lambda i: (i, 0)),
        out_shape=jax.ShapeDtypeStruct((_N, _FC_OUT), jnp.float32),
    )(m2, Wt, b2)


# speedup vs baseline: 1.4318x; 1.4318x over previous
import jax
import jax.numpy as jnp
import numpy as np
from jax.experimental import pallas as pl
from jax.experimental.pallas import tpu as pltpu

_NR = 8
_NW = 8
_H = 48
_NB = _NR * _NW
_N = 2048
_FC_OUT = 128
_BI = 16
_BR = 256


def _bin_indices(dx, dy):
    r = jnp.sqrt(dx * dx + dy * dy)
    ring = jnp.ceil((_NR - 1) * (jnp.log(r / 0.5) / 3.0))
    ring = jnp.where(jnp.isneginf(ring), 0.0, ring).astype(jnp.int32)

    ax = jnp.abs(dx)
    ay = jnp.abs(dy)
    ge = ay >= ax
    gt = ay > ax
    dxpos = dx > 0
    dxneg = dx < 0
    sbx = jnp.signbit(dx)
    sby = jnp.signbit(dy)
    w_pos = jnp.where(dxpos, jnp.where(ge, 4, 3), jnp.where(gt, 5, 6))
    w_neg = jnp.where(dxneg, 0, jnp.where(ay <= ax, 2, 1))
    w_zero = jnp.where(sbx, jnp.where(sby, -1, 7), 3)
    wedge = jnp.where(dy > 0, w_pos, jnp.where(dy < 0, w_neg, w_zero))

    fin = ring * _NW + wedge
    fin = jnp.where(fin < _NW, 0, fin)
    fin = jnp.where(fin >= _NW * _NW, 0, fin)
    return fin


def _means_step(yx_col, yy_col, yx_row, yy_row, hid, m_ref):
    dx = yx_row[...] - yx_col[...]
    dy = yy_row[...] - yy_col[...]
    fin = _bin_indices(dx, dy)

    k_iota = jax.lax.broadcasted_iota(jnp.int32, (_BI, _NB, _N), 1)
    onehot = (fin[:, None, :] == k_iota + _NW).astype(jnp.float32)
    onehot = onehot.reshape(_BI * _NB, _N)

    s = jnp.dot(onehot, hid[...], preferred_element_type=jnp.float32)
    cnt = s[:, _H:_H + 1]
    recip = 1.0 / jnp.maximum(cnt, 1.0)
    m_ref[...] = s[:, :_H] * recip


def _fc_step(m, Wt, b, out_ref):
    o = jnp.dot(m[...], Wt[...], preferred_element_type=jnp.float32) + b[...]
    out_ref[...] = jnp.maximum(o, 0.0)


def kernel(y_pred, x_start, hidden, W, b):
    del x_start
    yx_col = y_pred[:, 0:1]
    yy_col = y_pred[:, 1:2]
    yx_row = y_pred[:, 0].reshape(1, _N)
    yy_row = y_pred[:, 1].reshape(1, _N)
    hidden_aug = jnp.concatenate(
        [hidden, jnp.ones((_N, 1), hidden.dtype),
         jnp.zeros((_N, 64 - _H - 1), hidden.dtype)], axis=1)
    means = pl.pallas_call(
        _means_step,
        grid=(_N // _BI,),
        in_specs=[
            pl.BlockSpec((_BI, 1), lambda i: (i, 0)),
            pl.BlockSpec((_BI, 1), lambda i: (i, 0)),
            pl.BlockSpec((1, _N), lambda i: (0, 0)),
            pl.BlockSpec((1, _N), lambda i: (0, 0)),
            pl.BlockSpec((_N, 64), lambda i: (0, 0)),
        ],
        out_specs=pl.BlockSpec((_BI * _NB, _H), lambda i: (i, 0)),
        out_shape=jax.ShapeDtypeStruct((_N * _NB, _H), jnp.float32),
    )(yx_col, yy_col, yx_row, yy_row, hidden_aug)

    m2 = means.reshape(_N, _NB * _H)

    Wt = W.T
    b2 = b.reshape(1, _FC_OUT)
    return pl.pallas_call(
        _fc_step,
        grid=(_N // _BR,),
        in_specs=[
            pl.BlockSpec((_BR, _NB * _H), lambda i: (i, 0)),
            pl.BlockSpec((_NB * _H, _FC_OUT), lambda i: (0, 0)),
            pl.BlockSpec((1, _FC_OUT), lambda i: (0, 0)),
        ],
        out_specs=pl.BlockSpec((_BR, _FC_OUT), lambda i: (i, 0)),
        out_shape=jax.ShapeDtypeStruct((_N, _FC_OUT), jnp.float32),
    )(m2, Wt, b2)
